# baseline (speedup 1.0000x reference)
import functools

import numpy as np
import jax
import jax.numpy as jnp
from jax import lax
from jax.experimental import pallas as pl
from jax.experimental.pallas import tpu as pltpu
from jax.experimental.pallas import tpu_sc as plsc

B = 100000
D = 11
G = 10
LANES = 16

NUM_CORES = 2
NUM_SUBCORES = 16
NW = NUM_CORES * NUM_SUBCORES

CHUNK = 512
SC_CHUNKS = 2
SC_ROWS = NW * CHUNK * SC_CHUNKS
TC_ROWS = B - SC_ROWS
UNROLL = 4

BL = 2048
NB = -(-TC_ROWS // BL)


def _sq_thresholds():
    out = []
    for t in (0.1, 0.2, 0.3, 0.4, 0.5, 0.6, 0.7, 0.8, 0.9):
        t32 = np.float32(t)
        s = np.float32(t32 * t32)
        while np.float32(np.sqrt(np.nextafter(s, np.float32(0)))) >= t32:
            s = np.nextafter(s, np.float32(0))
        while np.float32(np.sqrt(s)) < t32:
            s = np.nextafter(s, np.float32(np.inf))
        out.append(float(s))
    return tuple(out)


_SQ_T = _sq_thresholds()


def _build_sc():
    mesh = plsc.VectorSubcoreMesh(core_axis_name="c", subcore_axis_name="s")

    @functools.partial(
        pl.kernel,
        out_type=jax.ShapeDtypeStruct((SC_ROWS,), jnp.float32),
        mesh=mesh,
        compiler_params=pltpu.CompilerParams(
            needs_layout_passes=False, skip_device_barrier=True),
        scratch_types=[
            pltpu.VMEM((8, CHUNK), jnp.float32),
            pltpu.VMEM((8, CHUNK), jnp.float32),
            pltpu.VMEM((8, CHUNK), jnp.float32),
            pltpu.VMEM((8, CHUNK), jnp.float32),
            pltpu.VMEM((G, CHUNK), jnp.float32),
            pltpu.VMEM((G, CHUNK), jnp.float32),
            pltpu.VMEM((CHUNK,), jnp.float32),
            pltpu.VMEM((CHUNK,), jnp.float32),
            pltpu.SemaphoreType.DMA,
            pltpu.SemaphoreType.DMA,
            pltpu.SemaphoreType.DMA,
            pltpu.SemaphoreType.DMA,
        ],
    )
    def k(z1_hbm, z2_hbm, dg_hbm, out_hbm,
          z1a, z1b, z2a, z2b, dga, dgb, outa, outb,
          isem_a, isem_b, osem_a, osem_b):
        z1buf = (z1a, z1b)
        z2buf = (z2a, z2b)
        dgbuf = (dga, dgb)
        outbuf = (outa, outb)
        isem = (isem_a, isem_b)
        osem = (osem_a, osem_b)

        wid = lax.axis_index("s") * NUM_CORES + lax.axis_index("c")
        lane = lax.iota(jnp.int32, LANES)

        def chunk_base(i):
            return pl.multiple_of((wid + i * NW) * CHUNK, 128)

        def in_copies(i):
            b = i % 2
            base = chunk_base(i)
            return (
                pltpu.make_async_copy(
                    z1_hbm.at[pl.ds(0, 8), pl.ds(base, CHUNK)],
                    z1buf[b], isem[b]),
                pltpu.make_async_copy(
                    z2_hbm.at[pl.ds(0, 8), pl.ds(base, CHUNK)],
                    z2buf[b], isem[b]),
                pltpu.make_async_copy(
                    dg_hbm.at[:, pl.ds(base, CHUNK)],
                    dgbuf[b], isem[b]),
            )

        def out_copy(i):
            b = i % 2
            return pltpu.make_async_copy(
                outbuf[b], out_hbm.at[pl.ds(chunk_base(i), CHUNK)], osem[b])

        def compute(i):
            b = i % 2
            for c in in_copies(i):
                c.wait()

            @plsc.parallel_loop(0, CHUNK // LANES, unroll=UNROLL)
            def _(v):
                off = v * LANES
                x1 = z1buf[b][0, pl.ds(off, LANES)]
                y1 = z1buf[b][2, pl.ds(off, LANES)]
                x2 = z2buf[b][0, pl.ds(off, LANES)]
                y2 = z2buf[b][2, pl.ds(off, LANES)]
                dx = x2 - x1
                dy = y2 - y1
                s = dx * dx + dy * dy
                did = jnp.zeros((LANES,), jnp.int32)
                for thr in _SQ_T:
                    did = did + (s >= jnp.float32(thr)).astype(jnp.int32)
                g = plsc.load_gather(dgbuf[b], [did, lane + off])
                outbuf[b][pl.ds(off, LANES)] = g

            out_copy(i).start()

        for c in in_copies(0):
            c.start()
        for i in range(SC_CHUNKS):
            if i + 1 < SC_CHUNKS:
                for c in in_copies(i + 1):
                    c.start()
            if i >= 2:
                out_copy(i - 2).wait()
            compute(i)
        for i in range(max(0, SC_CHUNKS - 2), SC_CHUNKS):
            out_copy(i).wait()

    return k


def _tc_body(z1_ref, z2_ref, dg_ref, out_ref):
    x1 = z1_ref[0, :]
    y1 = z1_ref[2, :]
    x2 = z2_ref[0, :]
    y2 = z2_ref[2, :]
    dx = x2 - x1
    dy = y2 - y1
    s = dx * dx + dy * dy

    def tree(lo, hi):
        if lo == hi:
            return dg_ref[lo, :]
        m = (lo + hi + 1) // 2
        return jnp.where(s >= jnp.float32(_SQ_T[m - 1]),
                         tree(m, hi), tree(lo, m - 1))

    out_ref[...] = tree(0, G - 1)


def _build_tc():
    off_blocks = SC_ROWS // BL
    return pl.pallas_call(
        _tc_body,
        grid=(NB,),
        in_specs=[
            pl.BlockSpec((8, BL), lambda i: (0, off_blocks + i)),
            pl.BlockSpec((8, BL), lambda i: (0, off_blocks + i)),
            pl.BlockSpec((G, BL), lambda i: (0, off_blocks + i)),
        ],
        out_specs=pl.BlockSpec((BL,), lambda i: (i,)),
        out_shape=jax.ShapeDtypeStruct((TC_ROWS,), jnp.float32),
    )


_sc_kernel = _build_sc()
_tc_kernel = _build_tc()


def kernel(z_1, z_2, dist_grade):
    z1t, z2t, dgt = z_1.T, z_2.T, dist_grade.T
    sc_out = _sc_kernel(z1t, z2t, dgt)
    tc_out = _tc_kernel(z1t, z2t, dgt)
    return jnp.concatenate([sc_out, tc_out])


# speedup vs baseline: 1.2201x; 1.2201x over previous
import functools

import numpy as np
import jax
import jax.numpy as jnp
from jax import lax
from jax.experimental import pallas as pl
from jax.experimental.pallas import tpu as pltpu
from jax.experimental.pallas import tpu_sc as plsc

B = 100000
D = 11
G = 10
LANES = 16

NUM_CORES = 2
NUM_SUBCORES = 16
NW = NUM_CORES * NUM_SUBCORES

CHUNK = 512
NFULL = 195
EXTRA_BASE = NFULL * CHUNK
EXTRA = 128
TAIL_BASE = EXTRA_BASE + EXTRA
TAIL = B - TAIL_BASE
MAX_CHUNKS = NFULL // NW + 1
UNROLL = 4

EXTRA_WID = NFULL - (MAX_CHUNKS - 1) * NW
TAIL_WID = EXTRA_WID + 1


def _sq_thresholds():
    out = []
    for t in (0.1, 0.2, 0.3, 0.4, 0.5, 0.6, 0.7, 0.8, 0.9):
        t32 = np.float32(t)
        s = np.float32(t32 * t32)
        while np.float32(np.sqrt(np.nextafter(s, np.float32(0)))) >= t32:
            s = np.nextafter(s, np.float32(0))
        while np.float32(np.sqrt(s)) < t32:
            s = np.nextafter(s, np.float32(np.inf))
        out.append(float(s))
    return tuple(out)


_SQ_T = _sq_thresholds()


def _build():
    mesh = plsc.VectorSubcoreMesh(core_axis_name="c", subcore_axis_name="s")

    @functools.partial(
        pl.kernel,
        out_type=jax.ShapeDtypeStruct((B,), jnp.float32),
        mesh=mesh,
        compiler_params=pltpu.CompilerParams(
            needs_layout_passes=False, skip_device_barrier=True),
        scratch_types=[
            pltpu.VMEM((8, CHUNK), jnp.float32),
            pltpu.VMEM((8, CHUNK), jnp.float32),
            pltpu.VMEM((8, CHUNK), jnp.float32),
            pltpu.VMEM((8, CHUNK), jnp.float32),
            pltpu.VMEM((G, CHUNK), jnp.float32),
            pltpu.VMEM((G, CHUNK), jnp.float32),
            pltpu.VMEM((CHUNK,), jnp.float32),
            pltpu.VMEM((CHUNK,), jnp.float32),
            pltpu.VMEM((8, TAIL), jnp.float32),
            pltpu.VMEM((8, TAIL), jnp.float32),
            pltpu.VMEM((G, TAIL), jnp.float32),
            pltpu.VMEM((TAIL,), jnp.float32),
            pltpu.SemaphoreType.DMA,
            pltpu.SemaphoreType.DMA,
            pltpu.SemaphoreType.DMA,
            pltpu.SemaphoreType.DMA,
        ],
    )
    def k(z1_hbm, z2_hbm, dg_hbm, out_hbm,
          z1a, z1b, z2a, z2b, dga, dgb, outa, outb,
          z1t, z2t, dgt, outt,
          isem_a, isem_b, osem_a, osem_b):
        z1buf = (z1a, z1b)
        z2buf = (z2a, z2b)
        dgbuf = (dga, dgb)
        outbuf = (outa, outb)
        isem = (isem_a, isem_b)
        osem = (osem_a, osem_b)

        wid = lax.axis_index("s") * NUM_CORES + lax.axis_index("c")
        lane = lax.iota(jnp.int32, LANES)

        def full_base(i):
            return pl.multiple_of((wid + i * NW) * CHUNK, 128)

        def in_copies(i):
            b = i % 2
            base = full_base(i)
            return (
                pltpu.make_async_copy(
                    z1_hbm.at[pl.ds(0, 8), pl.ds(base, CHUNK)],
                    z1buf[b], isem[b]),
                pltpu.make_async_copy(
                    z2_hbm.at[pl.ds(0, 8), pl.ds(base, CHUNK)],
                    z2buf[b], isem[b]),
                pltpu.make_async_copy(
                    dg_hbm.at[:, pl.ds(base, CHUNK)],
                    dgbuf[b], isem[b]),
            )

        def extra_in_copies():
            b = (MAX_CHUNKS - 1) % 2
            return (
                pltpu.make_async_copy(
                    z1_hbm.at[pl.ds(0, 8), pl.ds(EXTRA_BASE, EXTRA)],
                    z1buf[b].at[:, pl.ds(0, EXTRA)], isem[b]),
                pltpu.make_async_copy(
                    z2_hbm.at[pl.ds(0, 8), pl.ds(EXTRA_BASE, EXTRA)],
                    z2buf[b].at[:, pl.ds(0, EXTRA)], isem[b]),
                pltpu.make_async_copy(
                    dg_hbm.at[:, pl.ds(EXTRA_BASE, EXTRA)],
                    dgbuf[b].at[:, pl.ds(0, EXTRA)], isem[b]),
            )

        def tail_in_copies():
            b = (MAX_CHUNKS - 1) % 2
            return (
                pltpu.make_async_copy(
                    z1_hbm.at[pl.ds(0, 8), pl.ds(TAIL_BASE, TAIL)],
                    z1t, isem[b]),
                pltpu.make_async_copy(
                    z2_hbm.at[pl.ds(0, 8), pl.ds(TAIL_BASE, TAIL)],
                    z2t, isem[b]),
                pltpu.make_async_copy(
                    dg_hbm.at[:, pl.ds(TAIL_BASE, TAIL)],
                    dgt, isem[b]),
            )

        def out_copy(i):
            b = i % 2
            return pltpu.make_async_copy(
                outbuf[b], out_hbm.at[pl.ds(full_base(i), CHUNK)], osem[b])

        def extra_out_copy():
            b = (MAX_CHUNKS - 1) % 2
            return pltpu.make_async_copy(
                outbuf[b].at[pl.ds(0, EXTRA)],
                out_hbm.at[pl.ds(EXTRA_BASE, EXTRA)], osem[b])

        def tail_out_copy():
            b = (MAX_CHUNKS - 1) % 2
            return pltpu.make_async_copy(
                outt, out_hbm.at[pl.ds(TAIL_BASE, TAIL)], osem[b])

        def start_in(i):
            if i < MAX_CHUNKS - 1:
                for c in in_copies(i):
                    c.start()
            else:
                @pl.when(wid < NFULL - (MAX_CHUNKS - 1) * NW)
                def _():
                    for c in in_copies(i):
                        c.start()

                @pl.when(wid == EXTRA_WID)
                def _():
                    for c in extra_in_copies():
                        c.start()

                @pl.when(wid == TAIL_WID)
                def _():
                    for c in tail_in_copies():
                        c.start()

        def compute_group(z1r, z2r, dgr, outr, offs):
            xs = []
            for off in offs:
                x1 = z1r[0, pl.ds(off, LANES)]
                y1 = z1r[2, pl.ds(off, LANES)]
                x2 = z2r[0, pl.ds(off, LANES)]
                y2 = z2r[2, pl.ds(off, LANES)]
                xs.append((x2 - x1, y2 - y1))
            ss = [dx * dx + dy * dy for dx, dy in xs]
            dids = []
            for s in ss:
                did = jnp.zeros((LANES,), jnp.int32)
                for thr in _SQ_T:
                    did = did + (s >= jnp.float32(thr)).astype(jnp.int32)
                dids.append(did)
            for off, did in zip(offs, dids):
                g = plsc.load_gather(dgr, [did, lane + off])
                outr[pl.ds(off, LANES)] = g

        def compute(i):
            b = i % 2
            for c in in_copies(i):
                c.wait()

            @plsc.parallel_loop(0, CHUNK // LANES, unroll=UNROLL)
            def _(v):
                compute_group(z1buf[b], z2buf[b], dgbuf[b], outbuf[b],
                              [v * LANES])

            out_copy(i).start()

        def compute_extra():
            b = (MAX_CHUNKS - 1) % 2
            for c in extra_in_copies():
                c.wait()
            compute_group(z1buf[b], z2buf[b], dgbuf[b], outbuf[b],
                          [v * LANES for v in range(EXTRA // LANES)])
            extra_out_copy().start()

        def compute_tail():
            for c in tail_in_copies():
                c.wait()
            compute_group(z1t, z2t, dgt, outt,
                          [v * LANES for v in range(TAIL // LANES)])
            tail_out_copy().start()

        start_in(0)
        for i in range(MAX_CHUNKS):
            if i + 1 < MAX_CHUNKS:
                start_in(i + 1)
            if i >= 2:
                out_copy(i - 2).wait()
            if i < MAX_CHUNKS - 1:
                compute(i)
            else:
                pl.when(wid < NFULL - i * NW)(lambda: compute(i))
                pl.when(wid == EXTRA_WID)(compute_extra)
                pl.when(wid == TAIL_WID)(compute_tail)

        out_copy(MAX_CHUNKS - 2).wait()
        last = MAX_CHUNKS - 1
        pl.when(wid < NFULL - last * NW)(lambda: out_copy(last).wait())
        pl.when(wid == EXTRA_WID)(lambda: extra_out_copy().wait())
        pl.when(wid == TAIL_WID)(lambda: tail_out_copy().wait())

    return k


_sc_kernel = _build()


def kernel(z_1, z_2, dist_grade):
    return _sc_kernel(z_1.T, z_2.T, dist_grade.T)
